# baseline (speedup 1.0000x reference)
import functools

import jax
import jax.numpy as jnp
from jax import lax
from jax.experimental import pallas as pl
from jax.experimental.pallas import tpu as pltpu
from jax.experimental.pallas import tpu_sc as plsc

_B = 16384
_E = 64
_K = 10
_NT = 13
_G = _NT * _E
_R = 64
_WD = 1e-4
_BT = 256


def _sc_gather(idx_list, user_table, item_table, meta_tables):
    info = plsc.get_sparse_core_info()
    nc, ns = info.num_cores, info.num_subcores
    nw = nc * ns
    rpw = _B // nw
    nch = rpw // _R

    mesh = plsc.VectorSubcoreMesh(core_axis_name="c", subcore_axis_name="s")

    out_type = jax.ShapeDtypeStruct((_B, _G), jnp.float32)
    scratch_types = (
        [pltpu.VMEM((_R,), jnp.int32) for _ in range(_NT)]
        + [pltpu.VMEM((_R, _E), jnp.float32) for _ in range(_NT)]
        + [pltpu.SemaphoreType.DMA, pltpu.SemaphoreType.DMA]
    )

    @functools.partial(
        pl.kernel, mesh=mesh, out_type=out_type, scratch_types=scratch_types,
        compiler_params=pltpu.CompilerParams(use_tc_tiling_on_sc=False),
    )
    def sc_kernel(*refs):
        idx_h = refs[0:_NT]
        tbl_h = [refs[_NT], refs[_NT + 1], refs[_NT + 1]] + list(
            refs[_NT + 2:_NT + 2 + _K]
        )
        g_h = refs[_NT + 2 + _K]
        s = refs[_NT + 3 + _K:]
        idx_v = s[0:_NT]
        row_v = s[_NT:2 * _NT]
        sem_a, sem_b = s[2 * _NT], s[2 * _NT + 1]

        wid = lax.axis_index("s") * nc + lax.axis_index("c")
        base0 = wid * rpw

        def chunk(c, carry):
            base = pl.multiple_of(base0 + c * _R, _R)
            hs = [
                pltpu.async_copy(idx_h[i].at[pl.ds(base, _R)], idx_v[i], sem_a)
                for i in range(_NT)
            ]
            for h in hs:
                h.wait()
            gs = [
                pltpu.async_copy(tbl_h[t].at[idx_v[t]], row_v[t], sem_b)
                for t in range(_NT)
            ]
            for h in gs:
                h.wait()
            ws = [
                pltpu.async_copy(
                    row_v[t],
                    g_h.at[pl.ds(base, _R), pl.ds(t * _E, _E)],
                    sem_a,
                )
                for t in range(_NT)
            ]
            for h in ws:
                h.wait()
            return carry

        lax.fori_loop(0, nch, chunk, 0)

    return sc_kernel(*idx_list, user_table, item_table, *meta_tables)


def _tc_assemble(g):
    nblk = _B // _BT

    def body(g_ref, ue_ref, pe_ref, ne_ref, lp_ref, rg_ref):
        x = g_ref[...]
        u = x[:, 0:_E]
        p = x[:, _E:2 * _E]
        n = x[:, 2 * _E:3 * _E]
        ue_ref[:, 0, :] = u
        margin = (u * (p - n)) * jnp.float32(_K)
        ls = jnp.where(margin < 0, margin, jnp.float32(0.0)) - jnp.log1p(
            jnp.exp(-jnp.abs(margin))
        )
        lp = jnp.sum(ls)
        reg = jnp.sum(u * u)
        for k in range(_K):
            m = x[:, (3 + k) * _E:(4 + k) * _E]
            pe = p + m
            ne = n + m
            pe_ref[:, k, :] = pe
            ne_ref[:, k, :] = ne
            reg = reg + jnp.sum(pe * pe) + jnp.sum(ne * ne)

        @pl.when(pl.program_id(0) == 0)
        def _init():
            lp_ref[0, 0] = lp
            rg_ref[0, 0] = reg

        @pl.when(pl.program_id(0) != 0)
        def _accum():
            lp_ref[0, 0] += lp
            rg_ref[0, 0] += reg

    return pl.pallas_call(
        body,
        grid=(nblk,),
        in_specs=[pl.BlockSpec((_BT, _G), lambda i: (i, 0))],
        out_shape=[
            jax.ShapeDtypeStruct((_B, 1, _E), jnp.float32),
            jax.ShapeDtypeStruct((_B, _K, _E), jnp.float32),
            jax.ShapeDtypeStruct((_B, _K, _E), jnp.float32),
            jax.ShapeDtypeStruct((1, 1), jnp.float32),
            jax.ShapeDtypeStruct((1, 1), jnp.float32),
        ],
        out_specs=[
            pl.BlockSpec((_BT, 1, _E), lambda i: (i, 0, 0)),
            pl.BlockSpec((_BT, _K, _E), lambda i: (i, 0, 0)),
            pl.BlockSpec((_BT, _K, _E), lambda i: (i, 0, 0)),
            pl.BlockSpec(memory_space=pltpu.SMEM, index_map=lambda i: (0, 0)),
            pl.BlockSpec(memory_space=pltpu.SMEM, index_map=lambda i: (0, 0)),
        ],
    )(g)


def kernel(user, pos, prodcode, prodtype, graph_appear, colour_group, pcolval,
           pcolmas, depart, idxgroup, section, garmgroup, neg, history,
           history_mask, user_table, item_table, product_code_table,
           product_type_table, graphical_appearance_table, colour_group_table,
           perceived_colour_value_table, perceived_colour_master_table,
           department_table, index_group_table, section_table,
           garment_group_table):
    del history, history_mask

    def flat(i):
        return i.reshape(_B).astype(jnp.int32)

    idx_list = [flat(user), flat(pos), flat(neg), flat(prodcode),
                flat(prodtype), flat(graph_appear), flat(colour_group),
                flat(pcolval), flat(pcolmas), flat(depart), flat(idxgroup),
                flat(section), flat(garmgroup)]
    meta_tables = [product_code_table, product_type_table,
                   graphical_appearance_table, colour_group_table,
                   perceived_colour_value_table, perceived_colour_master_table,
                   department_table, index_group_table, section_table,
                   garment_group_table]

    g = _sc_gather(idx_list, user_table, item_table, meta_tables)
    u_rows, pos_embed, neg_embed, lp, rg = _tc_assemble(g)
    log_prob = lp[0, 0]
    reg = jnp.float32(_WD) * rg[0, 0]
    neg_log_prob = -log_prob
    loss = neg_log_prob + reg
    return (loss, neg_log_prob, reg, u_rows, pos_embed, neg_embed)


# speedup vs baseline: 1.3864x; 1.3864x over previous
import functools

import jax
import jax.numpy as jnp
from jax import lax
from jax.experimental import pallas as pl
from jax.experimental.pallas import tpu as pltpu
from jax.experimental.pallas import tpu_sc as plsc

_B = 16384
_E = 64
_K = 10
_NT = 13
_G = _NT * _E
_R = 64
_WD = 1e-4
_BT = 512


def _sc_gather(idx_list, user_table, item_table, meta_tables):
    info = plsc.get_sparse_core_info()
    nc, ns = info.num_cores, info.num_subcores
    nw = nc * ns
    rpw = _B // nw
    nch = rpw // _R

    mesh = plsc.VectorSubcoreMesh(core_axis_name="c", subcore_axis_name="s")

    out_type = jax.ShapeDtypeStruct((_B, _G), jnp.float32)
    scratch_types = (
        [pltpu.VMEM((rpw,), jnp.int32) for _ in range(_NT)]
        + [pltpu.VMEM((_R, _E), jnp.float32) for _ in range(2 * _NT)]
        + [pltpu.SemaphoreType.DMA, pltpu.SemaphoreType.DMA,
           pltpu.SemaphoreType.DMA, pltpu.SemaphoreType.DMA,
           pltpu.SemaphoreType.DMA]
    )

    @functools.partial(
        pl.kernel, mesh=mesh, out_type=out_type, scratch_types=scratch_types,
        compiler_params=pltpu.CompilerParams(use_tc_tiling_on_sc=False),
    )
    def sc_kernel(*refs):
        idx_h = refs[0:_NT]
        tbl_h = [refs[_NT], refs[_NT + 1], refs[_NT + 1]] + list(
            refs[_NT + 2:_NT + 2 + _K]
        )
        g_h = refs[_NT + 2 + _K]
        s = refs[_NT + 3 + _K:]
        idx_v = s[0:_NT]
        row_v = [s[_NT:2 * _NT], s[2 * _NT:3 * _NT]]
        sem_i = s[3 * _NT]
        sem_g = [s[3 * _NT + 1], s[3 * _NT + 2]]
        sem_w = [s[3 * _NT + 3], s[3 * _NT + 4]]

        wid = lax.axis_index("s") * nc + lax.axis_index("c")
        base0 = wid * rpw

        ihs = [
            pltpu.async_copy(idx_h[i].at[pl.ds(base0, rpw)], idx_v[i], sem_i)
            for i in range(_NT)
        ]
        for h in ihs:
            h.wait()

        def issue_gathers(c, b):
            return [
                pltpu.async_copy(
                    tbl_h[t].at[idx_v[t].at[pl.ds(c * _R, _R)]],
                    row_v[b][t],
                    sem_g[b],
                )
                for t in range(_NT)
            ]

        def issue_writes(c, b):
            base = pl.multiple_of(base0 + c * _R, _R)
            return [
                pltpu.async_copy(
                    row_v[b][t],
                    g_h.at[pl.ds(base, _R), pl.ds(t * _E, _E)],
                    sem_w[b],
                )
                for t in range(_NT)
            ]

        ghs = {0: issue_gathers(0, 0)}
        whs = {}
        for c in range(nch):
            b = c % 2
            nb = (c + 1) % 2
            if c + 1 < nch:
                if c - 1 >= 0:
                    for h in whs.pop(c - 1):
                        h.wait()
                ghs[c + 1] = issue_gathers(c + 1, nb)
            for h in ghs.pop(c):
                h.wait()
            whs[c] = issue_writes(c, b)
        for c in list(whs):
            for h in whs.pop(c):
                h.wait()

    return sc_kernel(*idx_list, user_table, item_table, *meta_tables)


def _tc_assemble(g):
    nblk = _B // _BT

    def body(g_ref, ue_ref, pe_ref, ne_ref, lp_ref, rg_ref):
        x = g_ref[...]
        u = x[:, 0:_E].T
        p = x[:, _E:2 * _E].T
        n = x[:, 2 * _E:3 * _E].T
        ue_ref[0, :, :] = u
        margin = (u * (p - n)) * jnp.float32(_K)
        ls = jnp.where(margin < 0, margin, jnp.float32(0.0)) - jnp.log1p(
            jnp.exp(-jnp.abs(margin))
        )
        lp = jnp.sum(ls)
        reg = jnp.sum(u * u)
        for k in range(_K):
            m = x[:, (3 + k) * _E:(4 + k) * _E].T
            pe = p + m
            ne = n + m
            pe_ref[k, :, :] = pe
            ne_ref[k, :, :] = ne
            reg = reg + jnp.sum(pe * pe) + jnp.sum(ne * ne)

        @pl.when(pl.program_id(0) == 0)
        def _init():
            lp_ref[0, 0] = lp
            rg_ref[0, 0] = reg

        @pl.when(pl.program_id(0) != 0)
        def _accum():
            lp_ref[0, 0] += lp
            rg_ref[0, 0] += reg

    return pl.pallas_call(
        body,
        grid=(nblk,),
        in_specs=[pl.BlockSpec((_BT, _G), lambda i: (i, 0))],
        out_shape=[
            jax.ShapeDtypeStruct((1, _E, _B), jnp.float32),
            jax.ShapeDtypeStruct((_K, _E, _B), jnp.float32),
            jax.ShapeDtypeStruct((_K, _E, _B), jnp.float32),
            jax.ShapeDtypeStruct((1, 1), jnp.float32),
            jax.ShapeDtypeStruct((1, 1), jnp.float32),
        ],
        out_specs=[
            pl.BlockSpec((1, _E, _BT), lambda i: (0, 0, i)),
            pl.BlockSpec((_K, _E, _BT), lambda i: (0, 0, i)),
            pl.BlockSpec((_K, _E, _BT), lambda i: (0, 0, i)),
            pl.BlockSpec(memory_space=pltpu.SMEM, index_map=lambda i: (0, 0)),
            pl.BlockSpec(memory_space=pltpu.SMEM, index_map=lambda i: (0, 0)),
        ],
    )(g)


def kernel(user, pos, prodcode, prodtype, graph_appear, colour_group, pcolval,
           pcolmas, depart, idxgroup, section, garmgroup, neg, history,
           history_mask, user_table, item_table, product_code_table,
           product_type_table, graphical_appearance_table, colour_group_table,
           perceived_colour_value_table, perceived_colour_master_table,
           department_table, index_group_table, section_table,
           garment_group_table):
    del history, history_mask

    def flat(i):
        return i.reshape(_B).astype(jnp.int32)

    idx_list = [flat(user), flat(pos), flat(neg), flat(prodcode),
                flat(prodtype), flat(graph_appear), flat(colour_group),
                flat(pcolval), flat(pcolmas), flat(depart), flat(idxgroup),
                flat(section), flat(garmgroup)]
    meta_tables = [product_code_table, product_type_table,
                   graphical_appearance_table, colour_group_table,
                   perceived_colour_value_table, perceived_colour_master_table,
                   department_table, index_group_table, section_table,
                   garment_group_table]

    g = _sc_gather(idx_list, user_table, item_table, meta_tables)
    ue_t, pe_t, ne_t, lp, rg = _tc_assemble(g)
    u_rows = jnp.transpose(ue_t, (2, 0, 1))
    pos_embed = jnp.transpose(pe_t, (2, 0, 1))
    neg_embed = jnp.transpose(ne_t, (2, 0, 1))
    log_prob = lp[0, 0]
    reg = jnp.float32(_WD) * rg[0, 0]
    neg_log_prob = -log_prob
    loss = neg_log_prob + reg
    return (loss, neg_log_prob, reg, u_rows, pos_embed, neg_embed)


# speedup vs baseline: 1.4044x; 1.0130x over previous
import functools

import jax
import jax.numpy as jnp
from jax import lax
from jax.experimental import pallas as pl
from jax.experimental.pallas import tpu as pltpu
from jax.experimental.pallas import tpu_sc as plsc

_B = 16384
_E = 64
_EP = 128
_K = 10
_NT = 13
_G = 896
_R = 32
_WD = 1e-4
_BT = 512


def _sc_gather(idx_list, user_table, item_table, meta_tables):
    info = plsc.get_sparse_core_info()
    nc, ns = info.num_cores, info.num_subcores
    nw = nc * ns
    rpw = _B // nw
    nch = rpw // _R

    mesh = plsc.VectorSubcoreMesh(core_axis_name="c", subcore_axis_name="s")

    out_type = jax.ShapeDtypeStruct((_B, _G), jnp.float32)
    scratch_types = (
        [pltpu.VMEM((rpw,), jnp.int32) for _ in range(_NT)]
        + [pltpu.VMEM((_R, _EP), jnp.float32) for _ in range(2 * _NT)]
        + [pltpu.SemaphoreType.DMA, pltpu.SemaphoreType.DMA,
           pltpu.SemaphoreType.DMA, pltpu.SemaphoreType.DMA,
           pltpu.SemaphoreType.DMA]
    )

    @functools.partial(
        pl.kernel, mesh=mesh, out_type=out_type, scratch_types=scratch_types,
        compiler_params=pltpu.CompilerParams(use_tc_tiling_on_sc=False),
    )
    def sc_kernel(*refs):
        idx_h = refs[0:_NT]
        tbl_h = [refs[_NT], refs[_NT + 1], refs[_NT + 1]] + list(
            refs[_NT + 2:_NT + 2 + _K]
        )
        g_h = refs[_NT + 2 + _K]
        s = refs[_NT + 3 + _K:]
        idx_v = s[0:_NT]
        row_v = [s[_NT:2 * _NT], s[2 * _NT:3 * _NT]]
        sem_i = s[3 * _NT]
        sem_g = [s[3 * _NT + 1], s[3 * _NT + 2]]
        sem_w = [s[3 * _NT + 3], s[3 * _NT + 4]]

        wid = lax.axis_index("s") * nc + lax.axis_index("c")
        base0 = wid * rpw

        ihs = [
            pltpu.async_copy(idx_h[i].at[pl.ds(base0, rpw)], idx_v[i], sem_i)
            for i in range(_NT)
        ]
        for h in ihs:
            h.wait()

        def issue_gathers(c, b):
            return [
                pltpu.async_copy(
                    tbl_h[t].at[idx_v[t].at[pl.ds(c * _R, _R)]],
                    row_v[b][t],
                    sem_g[b],
                )
                for t in range(_NT)
            ]

        def issue_writes(c, b):
            base = pl.multiple_of(base0 + c * _R, _R)
            return [
                pltpu.async_copy(
                    row_v[b][t].at[:, pl.ds(0, _E)],
                    g_h.at[pl.ds(base, _R), pl.ds(t * _E, _E)],
                    sem_w[b],
                )
                for t in range(_NT)
            ]

        ghs = {0: issue_gathers(0, 0)}
        whs = {}
        for c in range(nch):
            b = c % 2
            nb = (c + 1) % 2
            if c + 1 < nch:
                if c - 1 >= 0:
                    for h in whs.pop(c - 1):
                        h.wait()
                ghs[c + 1] = issue_gathers(c + 1, nb)
            for h in ghs.pop(c):
                h.wait()
            whs[c] = issue_writes(c, b)
        for c in list(whs):
            for h in whs.pop(c):
                h.wait()

    return sc_kernel(*idx_list, user_table, item_table, *meta_tables)


def _tc_assemble(g):
    nblk = _B // _BT

    def body(g_ref, ue_ref, pe_ref, ne_ref, lp_ref, rg_ref):
        x = g_ref[...]
        u = x[:, 0:_E].T
        p = x[:, _E:2 * _E].T
        n = x[:, 2 * _E:3 * _E].T
        ue_ref[0, :, :] = u
        margin = (u * (p - n)) * jnp.float32(_K)
        ls = jnp.where(margin < 0, margin, jnp.float32(0.0)) - jnp.log1p(
            jnp.exp(-jnp.abs(margin))
        )
        lp = jnp.sum(ls)
        reg = jnp.sum(u * u)
        for k in range(_K):
            m = x[:, (3 + k) * _E:(4 + k) * _E].T
            pe = p + m
            ne = n + m
            pe_ref[k, :, :] = pe
            ne_ref[k, :, :] = ne
            reg = reg + jnp.sum(pe * pe) + jnp.sum(ne * ne)

        @pl.when(pl.program_id(0) == 0)
        def _init():
            lp_ref[0, 0] = lp
            rg_ref[0, 0] = reg

        @pl.when(pl.program_id(0) != 0)
        def _accum():
            lp_ref[0, 0] += lp
            rg_ref[0, 0] += reg

    return pl.pallas_call(
        body,
        grid=(nblk,),
        in_specs=[pl.BlockSpec((_BT, _G), lambda i: (i, 0))],
        out_shape=[
            jax.ShapeDtypeStruct((1, _E, _B), jnp.float32),
            jax.ShapeDtypeStruct((_K, _E, _B), jnp.float32),
            jax.ShapeDtypeStruct((_K, _E, _B), jnp.float32),
            jax.ShapeDtypeStruct((1, 1), jnp.float32),
            jax.ShapeDtypeStruct((1, 1), jnp.float32),
        ],
        out_specs=[
            pl.BlockSpec((1, _E, _BT), lambda i: (0, 0, i)),
            pl.BlockSpec((_K, _E, _BT), lambda i: (0, 0, i)),
            pl.BlockSpec((_K, _E, _BT), lambda i: (0, 0, i)),
            pl.BlockSpec(memory_space=pltpu.SMEM, index_map=lambda i: (0, 0)),
            pl.BlockSpec(memory_space=pltpu.SMEM, index_map=lambda i: (0, 0)),
        ],
    )(g)


def kernel(user, pos, prodcode, prodtype, graph_appear, colour_group, pcolval,
           pcolmas, depart, idxgroup, section, garmgroup, neg, history,
           history_mask, user_table, item_table, product_code_table,
           product_type_table, graphical_appearance_table, colour_group_table,
           perceived_colour_value_table, perceived_colour_master_table,
           department_table, index_group_table, section_table,
           garment_group_table):
    del history, history_mask

    def flat(i):
        return i.reshape(_B).astype(jnp.int32)

    idx_list = [flat(user), flat(pos), flat(neg), flat(prodcode),
                flat(prodtype), flat(graph_appear), flat(colour_group),
                flat(pcolval), flat(pcolmas), flat(depart), flat(idxgroup),
                flat(section), flat(garmgroup)]

    def padt(t):
        return jnp.pad(t, ((0, 0), (0, _EP - _E)))

    user_table = padt(user_table)
    item_table = padt(item_table)
    meta_tables = [padt(t) for t in
                   [product_code_table, product_type_table,
                    graphical_appearance_table, colour_group_table,
                    perceived_colour_value_table,
                    perceived_colour_master_table, department_table,
                    index_group_table, section_table, garment_group_table]]

    g = _sc_gather(idx_list, user_table, item_table, meta_tables)
    ue_t, pe_t, ne_t, lp, rg = _tc_assemble(g)
    u_rows = jnp.transpose(ue_t, (2, 0, 1))
    pos_embed = jnp.transpose(pe_t, (2, 0, 1))
    neg_embed = jnp.transpose(ne_t, (2, 0, 1))
    log_prob = lp[0, 0]
    reg = jnp.float32(_WD) * rg[0, 0]
    neg_log_prob = -log_prob
    loss = neg_log_prob + reg
    return (loss, neg_log_prob, reg, u_rows, pos_embed, neg_embed)


# speedup vs baseline: 1.6675x; 1.1873x over previous
import functools

import jax
import jax.numpy as jnp
from jax import lax
from jax.experimental import pallas as pl
from jax.experimental.pallas import tpu as pltpu
from jax.experimental.pallas import tpu_sc as plsc

_B = 16384
_E = 64
_EP = 128
_K = 10
_NT = 13
_GS = 8
_R = 64
_WD = 1e-4
_BT = 512


def _sc_gather(idx_list, user_table, item_table, meta_tables):
    info = plsc.get_sparse_core_info()
    nc, ns = info.num_cores, info.num_subcores
    nw = nc * ns
    rpw = _B // nw
    nch = rpw // _R

    mesh = plsc.VectorSubcoreMesh(core_axis_name="c", subcore_axis_name="s")

    out_type = jax.ShapeDtypeStruct((_B, _GS, _EP), jnp.float32)
    scratch_types = (
        [pltpu.VMEM((rpw,), jnp.int32) for _ in range(_NT)]
        + [pltpu.VMEM((_R, _E), jnp.float32) for _ in range(2 * _NT)]
        + [pltpu.SemaphoreType.DMA, pltpu.SemaphoreType.DMA,
           pltpu.SemaphoreType.DMA, pltpu.SemaphoreType.DMA,
           pltpu.SemaphoreType.DMA]
    )

    @functools.partial(
        pl.kernel, mesh=mesh, out_type=out_type, scratch_types=scratch_types,
        compiler_params=pltpu.CompilerParams(use_tc_tiling_on_sc=False),
    )
    def sc_kernel(*refs):
        idx_h = refs[0:_NT]
        tbl_h = [refs[_NT], refs[_NT + 1], refs[_NT + 1]] + list(
            refs[_NT + 2:_NT + 2 + _K]
        )
        g_h = refs[_NT + 2 + _K]
        s = refs[_NT + 3 + _K:]
        idx_v = s[0:_NT]
        row_v = [s[_NT:2 * _NT], s[2 * _NT:3 * _NT]]
        sem_i = s[3 * _NT]
        sem_g = [s[3 * _NT + 1], s[3 * _NT + 2]]
        sem_w = [s[3 * _NT + 3], s[3 * _NT + 4]]

        wid = lax.axis_index("s") * nc + lax.axis_index("c")
        base0 = wid * rpw

        ihs = [
            pltpu.async_copy(idx_h[i].at[pl.ds(base0, rpw)], idx_v[i], sem_i)
            for i in range(_NT)
        ]
        for h in ihs:
            h.wait()

        def issue_gathers(c, b):
            return [
                pltpu.async_copy(
                    tbl_h[t].at[idx_v[t].at[pl.ds(c * _R, _R)]],
                    row_v[b][t],
                    sem_g[b],
                )
                for t in range(_NT)
            ]

        def issue_writes(c, b):
            base = pl.multiple_of(base0 + c * _R, _R)
            return [
                pltpu.async_copy(
                    row_v[b][t],
                    g_h.at[pl.ds(base, _R), t // 2, pl.ds((t % 2) * _E, _E)],
                    sem_w[b],
                )
                for t in range(_NT)
            ]

        ghs = {0: issue_gathers(0, 0)}
        whs = {}
        for c in range(nch):
            b = c % 2
            nb = (c + 1) % 2
            if c + 1 < nch:
                if c - 1 >= 0:
                    for h in whs.pop(c - 1):
                        h.wait()
                ghs[c + 1] = issue_gathers(c + 1, nb)
            for h in ghs.pop(c):
                h.wait()
            whs[c] = issue_writes(c, b)
        for c in list(whs):
            for h in whs.pop(c):
                h.wait()

    return sc_kernel(*idx_list, user_table, item_table, *meta_tables)


def _tc_assemble(g):
    nblk = _B // _BT

    def body(g_ref, ue_ref, pe_ref, ne_ref, lp_ref, rg_ref):
        x = g_ref[...]

        def slab(t):
            col = t * _E
            return x[:, col // _EP, (col % _EP):(col % _EP) + _E].T

        u = slab(0)
        p = slab(1)
        n = slab(2)
        ue_ref[0, :, :] = u
        margin = (u * (p - n)) * jnp.float32(_K)
        ls = jnp.where(margin < 0, margin, jnp.float32(0.0)) - jnp.log1p(
            jnp.exp(-jnp.abs(margin))
        )
        lp = jnp.sum(ls)
        reg = jnp.sum(u * u)
        for k in range(_K):
            m = slab(3 + k)
            pe = p + m
            ne = n + m
            pe_ref[k, :, :] = pe
            ne_ref[k, :, :] = ne
            reg = reg + jnp.sum(pe * pe) + jnp.sum(ne * ne)

        @pl.when(pl.program_id(0) == 0)
        def _init():
            lp_ref[0, 0] = lp
            rg_ref[0, 0] = reg

        @pl.when(pl.program_id(0) != 0)
        def _accum():
            lp_ref[0, 0] += lp
            rg_ref[0, 0] += reg

    return pl.pallas_call(
        body,
        grid=(nblk,),
        in_specs=[pl.BlockSpec((_BT, _GS, _EP), lambda i: (i, 0, 0))],
        out_shape=[
            jax.ShapeDtypeStruct((1, _E, _B), jnp.float32),
            jax.ShapeDtypeStruct((_K, _E, _B), jnp.float32),
            jax.ShapeDtypeStruct((_K, _E, _B), jnp.float32),
            jax.ShapeDtypeStruct((1, 1), jnp.float32),
            jax.ShapeDtypeStruct((1, 1), jnp.float32),
        ],
        out_specs=[
            pl.BlockSpec((1, _E, _BT), lambda i: (0, 0, i)),
            pl.BlockSpec((_K, _E, _BT), lambda i: (0, 0, i)),
            pl.BlockSpec((_K, _E, _BT), lambda i: (0, 0, i)),
            pl.BlockSpec(memory_space=pltpu.SMEM, index_map=lambda i: (0, 0)),
            pl.BlockSpec(memory_space=pltpu.SMEM, index_map=lambda i: (0, 0)),
        ],
    )(g)


def kernel(user, pos, prodcode, prodtype, graph_appear, colour_group, pcolval,
           pcolmas, depart, idxgroup, section, garmgroup, neg, history,
           history_mask, user_table, item_table, product_code_table,
           product_type_table, graphical_appearance_table, colour_group_table,
           perceived_colour_value_table, perceived_colour_master_table,
           department_table, index_group_table, section_table,
           garment_group_table):
    del history, history_mask

    def flat(i):
        return i.reshape(_B).astype(jnp.int32)

    def flat2(i):
        return flat(i) * 2

    idx_list = [flat2(user), flat2(pos), flat2(neg), flat2(prodcode),
                flat2(prodtype), flat2(graph_appear), flat2(colour_group),
                flat2(pcolval), flat2(pcolmas), flat2(depart), flat2(idxgroup),
                flat2(section), flat2(garmgroup)]

    def padt(t):
        n = t.shape[0]
        return jnp.pad(t, ((0, 0), (0, _EP - _E))).reshape(2 * n, _E)

    user_table = padt(user_table)
    item_table = padt(item_table)
    meta_tables = [padt(t) for t in
                   [product_code_table, product_type_table,
                    graphical_appearance_table, colour_group_table,
                    perceived_colour_value_table,
                    perceived_colour_master_table, department_table,
                    index_group_table, section_table, garment_group_table]]

    g = _sc_gather(idx_list, user_table, item_table, meta_tables)
    ue_t, pe_t, ne_t, lp, rg = _tc_assemble(g)
    u_rows = jnp.transpose(ue_t, (2, 0, 1))
    pos_embed = jnp.transpose(pe_t, (2, 0, 1))
    neg_embed = jnp.transpose(ne_t, (2, 0, 1))
    log_prob = lp[0, 0]
    reg = jnp.float32(_WD) * rg[0, 0]
    neg_log_prob = -log_prob
    loss = neg_log_prob + reg
    return (loss, neg_log_prob, reg, u_rows, pos_embed, neg_embed)
